# baseline (speedup 1.0000x reference)
import functools

import jax
import jax.numpy as jnp
from jax import lax
from jax.experimental import pallas as pl
from jax.experimental.pallas import tpu as pltpu
from jax.experimental.pallas import tpu_sc as plsc

N = 100000
D_EMB = 32
D_CBO = 16
D_ENC = 80
D_OUT = D_EMB + D_CBO + D_ENC

BLK = 128
SPAN = 3200
NBLK_MAIN = SPAN // BLK
NBLK_LAST = 6
TAIL_BASE = N - BLK
LAST_SPAN = N - (NW_ROWS := SPAN * 31)

_info = plsc.get_sparse_core_info()
NC = _info.num_cores
NS = _info.num_subcores
NW = NC * NS

_mesh = plsc.VectorSubcoreMesh(core_axis_name="c", subcore_axis_name="s")


@functools.partial(
    pl.kernel,
    mesh=_mesh,
    out_type=jax.ShapeDtypeStruct((N, D_OUT), jnp.float32),
    scratch_types=[
        pltpu.VMEM((SPAN,), jnp.int32),
        pltpu.VMEM((BLK, D_OUT), jnp.float32),
        pltpu.VMEM((BLK, D_OUT), jnp.float32),
        pltpu.VMEM((D_CBO, BLK), jnp.float32),
        pltpu.VMEM((D_CBO, BLK), jnp.float32),
        pltpu.VMEM((D_ENC, BLK), jnp.float32),
        pltpu.VMEM((D_ENC, BLK), jnp.float32),
        pltpu.SemaphoreType.DMA,
        pltpu.SemaphoreType.DMA,
        pltpu.SemaphoreType.DMA,
        pltpu.SemaphoreType.DMA,
        pltpu.SemaphoreType.DMA,
        pltpu.SemaphoreType.DMA,
        pltpu.SemaphoreType.DMA,
        pltpu.SemaphoreType.DMA,
    ],
    compiler_params=pltpu.CompilerParams(needs_layout_passes=False),
)
def _embed(gid, cboT, encT, table_pad, cboT_tail, encT_tail, out,
           idx_all, row0, row1, cbo0, cbo1, enc0, enc1,
           gs0, gs1, cs0, cs1, es0, es1, os0, os1):
    wid = lax.axis_index("s") * NC + lax.axis_index("c")
    base_w = wid * SPAN
    nblk = jnp.where(wid == NW - 1, NBLK_LAST, NBLK_MAIN)
    iota16 = lax.iota(jnp.int32, 16)

    rows = (row0, row1)
    cbos = (cbo0, cbo1)
    encs = (enc0, enc1)
    gss = (gs0, gs1)
    css = (cs0, cs1)
    ess = (es0, es1)
    oss = (os0, os1)

    def issue_inputs(t, p):
        base = base_w + t * BLK
        pltpu.async_copy(table_pad.at[idx_all.at[pl.ds(t * BLK, BLK)]],
                         rows[p], gss[p])
        pltpu.async_copy(cboT.at[:, pl.ds(base, BLK)], cbos[p], css[p])
        pltpu.async_copy(encT.at[:, pl.ds(base, BLK)], encs[p], ess[p])

    def wait_inputs(t, p):
        base = base_w + t * BLK
        pltpu.make_async_copy(table_pad.at[idx_all.at[pl.ds(t * BLK, BLK)]],
                              rows[p], gss[p]).wait()
        pltpu.make_async_copy(cboT.at[:, pl.ds(base, BLK)],
                              cbos[p], css[p]).wait()
        pltpu.make_async_copy(encT.at[:, pl.ds(base, BLK)],
                              encs[p], ess[p]).wait()

    def merge(p):
        row_v, cbo_v, enc_v = rows[p], cbos[p], encs[p]

        def mbody(g, _):
            off = g * 16
            rows16 = off + iota16
            for c in range(D_CBO):
                v = cbo_v[c, pl.ds(off, 16)]
                plsc.store_scatter(
                    row_v, [rows16, jnp.full((16,), D_EMB + c, jnp.int32)], v)
            for j in range(D_ENC):
                v = enc_v[j, pl.ds(off, 16)]
                plsc.store_scatter(
                    row_v,
                    [rows16, jnp.full((16,), D_EMB + D_CBO + j, jnp.int32)], v)
            return _

        lax.fori_loop(0, BLK // 16, mbody, None)

    def issue_out(t, p):
        pltpu.async_copy(rows[p], out.at[pl.ds(base_w + t * BLK, BLK)], oss[p])

    def wait_out(t, p):
        pltpu.make_async_copy(rows[p], out.at[pl.ds(base_w + t * BLK, BLK)],
                              oss[p]).wait()

    @pl.when(wid < NW - 1)
    def _():
        pltpu.sync_copy(gid.at[pl.ds(base_w, SPAN)], idx_all)

    @pl.when(wid == NW - 1)
    def _():
        pltpu.sync_copy(gid.at[pl.ds(base_w, LAST_SPAN)],
                        idx_all.at[pl.ds(0, LAST_SPAN)])

    issue_inputs(0, 0)
    issue_inputs(1, 1)

    def pair(u, _):
        for h in (0, 1):
            t = u * 2 + h

            @pl.when(t < nblk)
            def _():
                wait_inputs(t, h)
                merge(h)
                issue_out(t, h)

        for h in (0, 1):
            t_next = u * 2 + 2 + h

            @pl.when(t_next < nblk)
            def _():
                wait_out(t_next - 2, h)
                issue_inputs(t_next, h)

        return _

    lax.fori_loop(0, (NBLK_MAIN + 1) // 2, pair, None)

    @pl.when(wid < NW - 1)
    def _():
        wait_out(NBLK_MAIN - 2, (NBLK_MAIN - 2) % 2)
        wait_out(NBLK_MAIN - 1, (NBLK_MAIN - 1) % 2)

    @pl.when(wid == NW - 1)
    def _():
        wait_out(NBLK_LAST - 2, (NBLK_LAST - 2) % 2)
        wait_out(NBLK_LAST - 1, (NBLK_LAST - 1) % 2)

        toff = TAIL_BASE - base_w
        pltpu.async_copy(table_pad.at[idx_all.at[pl.ds(toff, BLK)]],
                         row0, gs0).wait()
        pltpu.sync_copy(cboT_tail, cbo0)
        pltpu.sync_copy(encT_tail, enc0)
        merge(0)
        pltpu.sync_copy(row0, out.at[pl.ds(TAIL_BASE, BLK)])


def kernel(op_gid, cbo, enc, op_table):
    table_pad = jnp.pad(op_table, ((0, 0), (0, D_OUT - D_EMB)))
    cboT = cbo.T
    encT = enc.T
    cboT_tail = cboT[:, TAIL_BASE:]
    encT_tail = encT[:, TAIL_BASE:]
    return _embed(op_gid.astype(jnp.int32), cboT, encT, table_pad,
                  cboT_tail, encT_tail)
